# Initial kernel scaffold; baseline (speedup 1.0000x reference)
import jax
import jax.numpy as jnp
from jax.experimental import pallas as pl


def kernel(x):
    raise NotImplementedError("write your pallas kernel here")


# speedup vs baseline: 2.1749x; 2.1749x over previous
import numpy as np
import jax
import jax.numpy as jnp
from jax.experimental import pallas as pl
from jax.experimental.pallas import tpu as pltpu

_BP = 16


def _haar_matrix() -> np.ndarray:
    i = np.arange(256)[:, None]
    o = np.arange(256)[None, :]
    s, ch = i // 64, i % 64
    c, k = o // 4, o % 4
    sign_b = 1 - 2 * (k // 2)
    sign_c = 1 - 2 * (k % 2)
    sign = np.where(s == 0, 1,
           np.where(s == 1, sign_b,
           np.where(s == 2, sign_c, sign_b * sign_c)))
    return (0.5 * sign * (ch == c)).astype(np.float32)


_M = _haar_matrix()


def _dwt_body(m_ref, x_ref, o_ref):
    xb = x_ref[...]
    top = xb[:, :256, :]
    bot = xb[:, 256:, :]
    xcat = jnp.concatenate([top, bot], axis=-1)
    x2 = xcat.reshape(_BP * 256, 256)
    o_ref[...] = jnp.dot(x2, m_ref[...], preferred_element_type=jnp.float32)


def kernel(x):
    B, H, W, C = x.shape
    RP = B * H // 2
    xv = x.reshape(RP, 2 * (W // 2), 2 * C)
    n = RP // _BP // 2

    out = pl.pallas_call(
        _dwt_body,
        out_shape=jax.ShapeDtypeStruct((RP * (W // 2), 4 * C), jnp.float32),
        grid=(2, n),
        in_specs=[
            pl.BlockSpec((256, 256), lambda c, i: (0, 0)),
            pl.BlockSpec((_BP, 2 * (W // 2), 2 * C),
                         lambda c, i, _n=n: (c * _n + i, 0, 0)),
        ],
        out_specs=pl.BlockSpec((_BP * (W // 2), 4 * C),
                               lambda c, i, _n=n: (c * _n + i, 0)),
        compiler_params=pltpu.CompilerParams(
            dimension_semantics=("parallel", "arbitrary"),
        ),
        name="dwt_pool",
    )(jnp.asarray(_M), xv)
    return out.reshape(B, H // 2, W // 2, 4 * C)


# speedup vs baseline: 3.7340x; 1.7169x over previous
import numpy as np
import jax
import jax.numpy as jnp
from jax.experimental import pallas as pl
from jax.experimental.pallas import tpu as pltpu

_BP = 16


def _perm_matrix() -> np.ndarray:
    p = np.zeros((256, 256), np.float32)
    for k in range(4):
        for c in range(64):
            p[64 * k + c, 4 * c + k] = 1.0
    return p


_P = _perm_matrix()


def _dwt_body(p_ref, x_ref, o_ref):
    a = x_ref[0::2, 0::2, :]
    b = x_ref[0::2, 1::2, :]
    c = x_ref[1::2, 0::2, :]
    d = x_ref[1::2, 1::2, :]
    s, t = a + b, a - b
    u_, v_ = c + d, c - d
    ll = 0.5 * (s + u_)
    lh = 0.5 * (s - u_)
    hl = 0.5 * (t + v_)
    hh = 0.5 * (t - v_)
    u = jnp.concatenate([ll, lh, hl, hh], axis=-1)
    u2 = u.reshape(_BP * 256, 256)
    o = jnp.dot(u2, p_ref[...], preferred_element_type=jnp.float32)
    o_ref[...] = o.reshape(_BP, 256, 256)


def kernel(x):
    B, H, W, C = x.shape
    xm = x.reshape(B * H, W, C)
    RP = B * H // 2
    n = RP // _BP // 2

    out = pl.pallas_call(
        _dwt_body,
        out_shape=jax.ShapeDtypeStruct((RP, W // 2, 4 * C), jnp.float32),
        grid=(2, n),
        in_specs=[
            pl.BlockSpec((256, 256), lambda c, i: (0, 0)),
            pl.BlockSpec((2 * _BP, W, C),
                         lambda c, i, _n=n: (c * _n + i, 0, 0)),
        ],
        out_specs=pl.BlockSpec((_BP, W // 2, 4 * C),
                               lambda c, i, _n=n: (c * _n + i, 0, 0)),
        compiler_params=pltpu.CompilerParams(
            dimension_semantics=("parallel", "arbitrary"),
        ),
        name="dwt_pool",
    )(jnp.asarray(_P), xm)
    return out.reshape(B, H // 2, W // 2, 4 * C)
